# baseline (speedup 1.0000x reference)
import functools

import jax
import jax.numpy as jnp
from jax import lax
from jax.experimental import pallas as pl
from jax.experimental.pallas import tpu as pltpu
from jax.experimental.pallas import tpu_sc as plsc

DIM = 768
K = 128
M = 20000
HID = 150
DEMB = 20
CPROP = 2

IBLK = 32
NBLK = K // IBLK

NWORK = 32
ROWS_PER_W = M // NWORK
_PREC = lax.Precision.DEFAULT


def _coref_body(u_ref, sb_ref, se_ref, tril_ref, ssc_ref, ssr_ref,
                idxc_ref, idxr_ref, idxn_ref,
                wl_ref, wr_ref, wp_ref, dtp_ref, wd_ref, woc_ref, ball_ref,
                bout_ref, wg1_ref, wg2_ref, bg_ref,
                upd_out, sc_out, sval_out,
                u16_scr, l16_scr, r16_scr, scc_scr, bk_scr, dt16_scr, wp16_scr):
    f32 = jnp.float32
    bf16 = jnp.bfloat16

    d = jnp.maximum(sb_ref[...] - se_ref[...], 0)
    blog = (5 + (d >= 8).astype(jnp.int32) + (d >= 16).astype(jnp.int32)
            + (d >= 32).astype(jnp.int32) + (d >= 64).astype(jnp.int32))
    bk_scr[...] = jnp.where(d <= 4, d, blog)

    dw = jnp.dot(dtp_ref[...], wd_ref[...], precision=_PREC)
    row = lax.broadcasted_iota(jnp.int32, (16, HID), 0)
    dw = jnp.where(row == 15, ball_ref[...], dw).astype(bf16)
    wp16_scr[...] = wp_ref[...].astype(bf16)
    woc16 = woc_ref[...].astype(bf16)
    is15 = lax.broadcasted_iota(jnp.int32, (1, 1, 16), 2) == 15

    def dtblk(b, carry):
        r0 = pl.multiple_of(b * IBLK, IBLK)
        bkt = bk_scr[pl.ds(r0, IBLK), :]
        oh = jnp.logical_or(
            bkt[:, :, None] == lax.broadcasted_iota(jnp.int32, (1, 1, 16), 2),
            is15).astype(bf16)
        dt = jnp.dot(oh.reshape(IBLK * K, 16), dw,
                     preferred_element_type=f32)
        dt16_scr[pl.ds(r0 * K, IBLK * K), :] = dt.astype(bf16)
        return carry

    lax.fori_loop(0, NBLK, dtblk, 0)

    def pair_scores(u):
        u16 = u.astype(bf16)
        u16_scr[...] = u16
        l16_scr[...] = jnp.dot(u, wl_ref[...], precision=_PREC).astype(bf16)
        r16_scr[...] = jnp.dot(u, wr_ref[...], precision=_PREC).astype(bf16)

        def blk(b, carry):
            r0 = pl.multiple_of(b * IBLK, IBLK)
            ub16 = u16_scr[pl.ds(r0, IBLK), :]
            pairs = (ub16[:, None, :] * u16[None, :, :]).reshape(IBLK * K, DIM)
            prod = jnp.dot(pairs, wp16_scr[...],
                           preferred_element_type=f32)
            h = (prod.astype(bf16) + dt16_scr[pl.ds(r0 * K, IBLK * K), :])
            h = (h.reshape(IBLK, K, HID) + l16_scr[pl.ds(r0, IBLK), :][:, None, :]
                 + r16_scr[...][None, :, :])
            h = jnp.maximum(h, jnp.array(0.0, bf16)).reshape(IBLK * K, HID)
            scc_scr[pl.ds(r0 * K, IBLK * K), :] = jnp.dot(
                h, woc16, preferred_element_type=f32)
            return carry

        lax.fori_loop(0, NBLK, blk, 0)
        return scc_scr[...].reshape(K, K) + bout_ref[...]

    u0 = u_ref[...]
    scores = pair_scores(u0)
    scores = scores + ssc_ref[...] + ssr_ref[...]
    ii = lax.broadcasted_iota(jnp.int32, (K, K), 0)
    jj = lax.broadcasted_iota(jnp.int32, (K, K), 1)
    scores = jnp.where(ii == jj, 0.0, scores)

    neg = (1.0 - tril_ref[...]) * 1e23

    def round_body(t, uv_scores):
        u, scores = uv_scores
        s2 = scores - neg
        s2 = s2 - jnp.max(s2, axis=-1, keepdims=True)
        e = jnp.exp(s2)
        probs = e / jnp.sum(e, axis=-1, keepdims=True)
        ctxt = jnp.dot(probs, u, precision=_PREC)
        g = jax.nn.sigmoid(jnp.dot(u, wg1_ref[...], precision=_PREC)
                           + jnp.dot(ctxt, wg2_ref[...], precision=_PREC)
                           + bg_ref[...])
        u = g * u + (1.0 - g) * ctxt
        return (u, pair_scores(u))

    u, scores = lax.fori_loop(0, CPROP, round_body, (u0, scores))

    upd_out[...] = u
    sc_out[...] = scores
    sel = ((idxc_ref[...] == idxr_ref[...]) & (idxr_ref[...] != idxn_ref[...])).astype(f32)
    sval_out[...] = jnp.dot(sel, u, precision=_PREC)


CH = 64
CHSHIFT = 6
NFULL = M // CH
CPW = -(-NFULL // NWORK)


def _scatter_body(cand_hbm, svals_hbm, idx_hbm, len_hbm, out_hbm,
                  bufa, bufb, idx_v, oik_v, ok_v, len_v, sv_sh,
                  sia, sib, soa, sob):
    wid = lax.axis_index("s") * 2 + lax.axis_index("c")
    sid = lax.axis_index("s")

    @pl.when(sid == 0)
    def _():
        pltpu.sync_copy(svals_hbm, sv_sh)

    pltpu.sync_copy(idx_hbm, idx_v.at[pl.ds(0, K)])
    pltpu.sync_copy(len_hbm, len_v)
    nvalid = len_v[...][0]

    def kbody(k, off):
        ik = idx_v[pl.ds(k, 16)][0]
        owned = jnp.logical_and((ik >> CHSHIFT) & (NWORK - 1) == wid, k < nvalid)

        @pl.when(owned)
        def _():
            oik_v[pl.ds(off, 16)] = lax.broadcast(ik, (16,))
            ok_v[pl.ds(off, 16)] = lax.broadcast(k, (16,))

        return off + jnp.where(owned, 1, 0)

    n_owned = lax.fori_loop(0, K, kbody, 0)

    plsc.subcore_barrier()

    bufs = (bufa, bufb)
    sin = (sia, sib)
    sout = (soa, sob)

    def cid_of(c):
        return wid + NWORK * c

    def in_d(c):
        p = c & 1
        return pltpu.make_async_copy(
            cand_hbm.at[pl.ds(cid_of(c) * CH, CH)], bufs[p], sin[p])

    def out_d(c):
        p = c & 1
        return pltpu.make_async_copy(
            bufs[p], out_hbm.at[pl.ds(cid_of(c) * CH, CH)], sout[p])

    def merge(buf, start, size):
        def jbody(j, carry):
            ikj = oik_v[pl.ds(j, 16)][0]
            kj = ok_v[pl.ds(j, 16)][0]

            @pl.when((ikj >= start) & (ikj < start + size))
            def _():
                pltpu.sync_copy(sv_sh.at[pl.ds(kj, 1)],
                                buf.at[pl.ds(ikj - start, 1)])

            return carry

        lax.fori_loop(0, n_owned, jbody, 0)

    @pl.when(cid_of(0) < NFULL)
    def _():
        in_d(0).start()

    for c in range(CPW):
        if c + 1 < CPW:
            @pl.when(cid_of(c + 1) < NFULL)
            def _(c=c):
                if c >= 1:
                    out_d(c - 1).wait()
                in_d(c + 1).start()

        @pl.when(cid_of(c) < NFULL)
        def _(c=c):
            in_d(c).wait()
            merge(bufs[c & 1], cid_of(c) * CH, CH)
            out_d(c).start()

    for c in range(CPW):
        @pl.when(jnp.logical_and(cid_of(c) < NFULL,
                                 cid_of(c) + 2 * NWORK >= NFULL))
        def _(c=c):
            out_d(c).wait()

    @pl.when(wid == NFULL % NWORK)
    def _():
        tstart = NFULL * CH
        tsize = M - NFULL * CH
        pltpu.sync_copy(cand_hbm.at[pl.ds(tstart, tsize)],
                        bufa.at[pl.ds(0, tsize)])
        merge(bufa, tstart, tsize)
        pltpu.sync_copy(bufa.at[pl.ds(0, tsize)],
                        out_hbm.at[pl.ds(tstart, tsize)])


def kernel(cand_span_vecs, span_vecs, span_begin, span_end, triangular_mask,
           span_scores, prune_indices_hoi, span_lengths, W_left, b_left,
           W_right, b_right, W_prod, b_prod, dist_table, W_dist, b_dist,
           W_out, b_out, W_gate, b_gate):
    f32 = jnp.float32
    u0 = span_vecs[0]
    sb = span_begin.reshape(K, 1)
    se = span_end.reshape(1, K)
    tril = triangular_mask[0]
    ssc = span_scores.reshape(K, 1)
    ssr = span_scores.reshape(1, K)
    idx = prune_indices_hoi[0]
    idxc = idx.reshape(K, 1)
    idxr = idx.reshape(1, K)
    idxn = jnp.concatenate([idx[1:], jnp.full((1,), -1, idx.dtype)]).reshape(1, K)
    dtp = jnp.concatenate([dist_table, jnp.zeros((16 - dist_table.shape[0], DEMB), f32)], axis=0)
    ball = (b_left + b_right + b_prod + b_dist).reshape(1, HID)
    bout = b_out.reshape(1, 1)
    wg1 = W_gate[:DIM]
    wg2 = W_gate[DIM:]
    bg = b_gate.reshape(1, DIM)

    upd, scores, svals = pl.pallas_call(
        _coref_body,
        out_shape=(
            jax.ShapeDtypeStruct((K, DIM), f32),
            jax.ShapeDtypeStruct((K, K), f32),
            jax.ShapeDtypeStruct((K, DIM), f32),
        ),
        scratch_shapes=[
            pltpu.VMEM((K, DIM), jnp.bfloat16),
            pltpu.VMEM((K, HID), jnp.bfloat16),
            pltpu.VMEM((K, HID), jnp.bfloat16),
            pltpu.VMEM((K * K, 1), f32),
            pltpu.VMEM((K, K), jnp.int32),
            pltpu.VMEM((K * K, HID), jnp.bfloat16),
            pltpu.VMEM((DIM, HID), jnp.bfloat16),
        ],
    )(u0, sb, se, tril, ssc, ssr, idxc, idxr, idxn,
      W_left, W_right, W_prod, dtp, W_dist, W_out, ball, bout, wg1, wg2, bg)

    lens = jnp.broadcast_to(span_lengths.astype(jnp.int32), (16,))
    mesh = plsc.VectorSubcoreMesh(core_axis_name="c", subcore_axis_name="s",
                                  num_cores=2, num_subcores=16)
    scatter = functools.partial(
        pl.kernel,
        out_type=jax.ShapeDtypeStruct((M, DIM), f32),
        mesh=mesh,
        scratch_types=[
            pltpu.VMEM((CH, DIM), f32),
            pltpu.VMEM((CH, DIM), f32),
            pltpu.VMEM((K + 16,), jnp.int32),
            pltpu.VMEM((K + 16,), jnp.int32),
            pltpu.VMEM((K + 16,), jnp.int32),
            pltpu.VMEM((16,), jnp.int32),
            pltpu.VMEM_SHARED((K, DIM), f32),
            pltpu.SemaphoreType.DMA,
            pltpu.SemaphoreType.DMA,
            pltpu.SemaphoreType.DMA,
            pltpu.SemaphoreType.DMA,
        ],
    )(_scatter_body)
    new_cand = scatter(cand_span_vecs[0], svals, idx, lens)

    return (new_cand[None], upd[None], scores[None])


# speedup vs baseline: 1.1069x; 1.1069x over previous
import functools

import jax
import jax.numpy as jnp
from jax import lax
from jax.experimental import pallas as pl
from jax.experimental.pallas import tpu as pltpu
from jax.experimental.pallas import tpu_sc as plsc

DIM = 768
K = 128
M = 20000
HID = 150
DEMB = 20
CPROP = 2

IBLK = 32
NBLK = K // IBLK

NWORK = 32
ROWS_PER_W = M // NWORK
_PREC = lax.Precision.DEFAULT


def _coref_body(u_ref, sb_ref, se_ref, tril_ref, ssc_ref, ssr_ref,
                idxc_ref, idxr_ref, idxn_ref,
                wl_ref, wr_ref, wp_ref, dtp_ref, wd_ref, woc_ref, ball_ref,
                bout_ref, wg1_ref, wg2_ref, bg_ref,
                upd_out, sc_out, sval_out,
                u16_scr, l16_scr, r16_scr, scc_scr, bk_scr, dt16_scr, wp16_scr):
    f32 = jnp.float32
    bf16 = jnp.bfloat16

    d = jnp.maximum(sb_ref[...] - se_ref[...], 0)
    blog = (5 + (d >= 8).astype(jnp.int32) + (d >= 16).astype(jnp.int32)
            + (d >= 32).astype(jnp.int32) + (d >= 64).astype(jnp.int32))
    bk_scr[...] = jnp.where(d <= 4, d, blog)

    dw = jnp.dot(dtp_ref[...], wd_ref[...], precision=_PREC)
    row = lax.broadcasted_iota(jnp.int32, (16, HID), 0)
    dw = jnp.where(row == 15, ball_ref[...], dw).astype(bf16)
    wp16_scr[...] = wp_ref[...].astype(bf16)
    wo16 = woc_ref[...].astype(bf16)
    is15 = lax.broadcasted_iota(jnp.int32, (1, 1, 16), 2) == 15

    def dtblk(b, carry):
        r0 = pl.multiple_of(b * IBLK, IBLK)
        bkt = bk_scr[pl.ds(r0, IBLK), :]
        oh = jnp.logical_or(
            bkt[:, :, None] == lax.broadcasted_iota(jnp.int32, (1, 1, 16), 2),
            is15).astype(bf16)
        dt = jnp.dot(oh.reshape(IBLK * K, 16), dw,
                     preferred_element_type=f32)
        dt16_scr[pl.ds(r0 * K, IBLK * K), :] = dt.astype(bf16)
        return carry

    lax.fori_loop(0, NBLK, dtblk, 0)

    def pair_scores(u):
        u16 = u.astype(bf16)
        u16_scr[...] = u16
        l16_scr[...] = jnp.dot(u, wl_ref[...], precision=_PREC).astype(bf16)
        r16_scr[...] = jnp.dot(u, wr_ref[...], precision=_PREC).astype(bf16)

        def blk(b, carry):
            r0 = pl.multiple_of(b * IBLK, IBLK)
            ub16 = u16_scr[pl.ds(r0, IBLK), :]
            pairs = (ub16[:, None, :] * u16[None, :, :]).reshape(IBLK * K, DIM)
            prod = jnp.dot(pairs, wp16_scr[...],
                           preferred_element_type=f32)
            h = (prod.astype(bf16) + dt16_scr[pl.ds(r0 * K, IBLK * K), :])
            h = (h.reshape(IBLK, K, HID) + l16_scr[pl.ds(r0, IBLK), :][:, None, :]
                 + r16_scr[...][None, :, :])
            h = jnp.maximum(h, jnp.array(0.0, bf16))
            scc_scr[pl.ds(r0, IBLK), :] = jnp.sum(
                (h * wo16[None, :, :]).astype(f32), axis=-1)
            return carry

        lax.fori_loop(0, NBLK, blk, 0)
        return scc_scr[...] + bout_ref[...]

    u0 = u_ref[...]
    scores = pair_scores(u0)
    scores = scores + ssc_ref[...] + ssr_ref[...]
    ii = lax.broadcasted_iota(jnp.int32, (K, K), 0)
    jj = lax.broadcasted_iota(jnp.int32, (K, K), 1)
    scores = jnp.where(ii == jj, 0.0, scores)

    neg = (1.0 - tril_ref[...]) * 1e23

    def round_body(t, uv_scores):
        u, scores = uv_scores
        s2 = scores - neg
        s2 = s2 - jnp.max(s2, axis=-1, keepdims=True)
        e = jnp.exp(s2)
        probs = e / jnp.sum(e, axis=-1, keepdims=True)
        ctxt = jnp.dot(probs, u, precision=_PREC)
        g = jax.nn.sigmoid(jnp.dot(u, wg1_ref[...], precision=_PREC)
                           + jnp.dot(ctxt, wg2_ref[...], precision=_PREC)
                           + bg_ref[...])
        u = g * u + (1.0 - g) * ctxt
        return (u, pair_scores(u))

    u, scores = lax.fori_loop(0, CPROP, round_body, (u0, scores))

    upd_out[...] = u
    sc_out[...] = scores
    sel = ((idxc_ref[...] == idxr_ref[...]) & (idxr_ref[...] != idxn_ref[...])).astype(f32)
    sval_out[...] = jnp.dot(sel, u, precision=_PREC)


CH = 64
CHSHIFT = 6
NFULL = M // CH
CPW = -(-NFULL // NWORK)


def _scatter_body(cand_hbm, svals_hbm, idx_hbm, len_hbm, out_hbm,
                  bufa, bufb, idx_v, oik_v, ok_v, len_v, sv_sh,
                  sia, sib, soa, sob):
    wid = lax.axis_index("s") * 2 + lax.axis_index("c")
    sid = lax.axis_index("s")

    @pl.when(sid == 0)
    def _():
        pltpu.sync_copy(svals_hbm, sv_sh)

    pltpu.sync_copy(idx_hbm, idx_v.at[pl.ds(0, K)])
    pltpu.sync_copy(len_hbm, len_v)
    nvalid = len_v[...][0]

    def kbody(k, off):
        ik = idx_v[pl.ds(k, 16)][0]
        owned = jnp.logical_and((ik >> CHSHIFT) & (NWORK - 1) == wid, k < nvalid)

        @pl.when(owned)
        def _():
            oik_v[pl.ds(off, 16)] = lax.broadcast(ik, (16,))
            ok_v[pl.ds(off, 16)] = lax.broadcast(k, (16,))

        return off + jnp.where(owned, 1, 0)

    n_owned = lax.fori_loop(0, K, kbody, 0)

    plsc.subcore_barrier()

    bufs = (bufa, bufb)
    sin = (sia, sib)
    sout = (soa, sob)

    def cid_of(c):
        return wid + NWORK * c

    def in_d(c):
        p = c & 1
        return pltpu.make_async_copy(
            cand_hbm.at[pl.ds(cid_of(c) * CH, CH)], bufs[p], sin[p])

    def out_d(c):
        p = c & 1
        return pltpu.make_async_copy(
            bufs[p], out_hbm.at[pl.ds(cid_of(c) * CH, CH)], sout[p])

    def merge(buf, start, size):
        def jbody(j, carry):
            ikj = oik_v[pl.ds(j, 16)][0]
            kj = ok_v[pl.ds(j, 16)][0]

            @pl.when((ikj >= start) & (ikj < start + size))
            def _():
                pltpu.sync_copy(sv_sh.at[pl.ds(kj, 1)],
                                buf.at[pl.ds(ikj - start, 1)])

            return carry

        lax.fori_loop(0, n_owned, jbody, 0)

    @pl.when(cid_of(0) < NFULL)
    def _():
        in_d(0).start()

    for c in range(CPW):
        if c + 1 < CPW:
            @pl.when(cid_of(c + 1) < NFULL)
            def _(c=c):
                if c >= 1:
                    out_d(c - 1).wait()
                in_d(c + 1).start()

        @pl.when(cid_of(c) < NFULL)
        def _(c=c):
            in_d(c).wait()
            merge(bufs[c & 1], cid_of(c) * CH, CH)
            out_d(c).start()

    for c in range(CPW):
        @pl.when(jnp.logical_and(cid_of(c) < NFULL,
                                 cid_of(c) + 2 * NWORK >= NFULL))
        def _(c=c):
            out_d(c).wait()

    @pl.when(wid == NFULL % NWORK)
    def _():
        tstart = NFULL * CH
        tsize = M - NFULL * CH
        pltpu.sync_copy(cand_hbm.at[pl.ds(tstart, tsize)],
                        bufa.at[pl.ds(0, tsize)])
        merge(bufa, tstart, tsize)
        pltpu.sync_copy(bufa.at[pl.ds(0, tsize)],
                        out_hbm.at[pl.ds(tstart, tsize)])


def kernel(cand_span_vecs, span_vecs, span_begin, span_end, triangular_mask,
           span_scores, prune_indices_hoi, span_lengths, W_left, b_left,
           W_right, b_right, W_prod, b_prod, dist_table, W_dist, b_dist,
           W_out, b_out, W_gate, b_gate):
    f32 = jnp.float32
    u0 = span_vecs[0]
    sb = span_begin.reshape(K, 1)
    se = span_end.reshape(1, K)
    tril = triangular_mask[0]
    ssc = span_scores.reshape(K, 1)
    ssr = span_scores.reshape(1, K)
    idx = prune_indices_hoi[0]
    idxc = idx.reshape(K, 1)
    idxr = idx.reshape(1, K)
    idxn = jnp.concatenate([idx[1:], jnp.full((1,), -1, idx.dtype)]).reshape(1, K)
    dtp = jnp.concatenate([dist_table, jnp.zeros((16 - dist_table.shape[0], DEMB), f32)], axis=0)
    ball = (b_left + b_right + b_prod + b_dist).reshape(1, HID)
    wo = W_out.reshape(1, HID)
    bout = b_out.reshape(1, 1)
    wg1 = W_gate[:DIM]
    wg2 = W_gate[DIM:]
    bg = b_gate.reshape(1, DIM)

    upd, scores, svals = pl.pallas_call(
        _coref_body,
        out_shape=(
            jax.ShapeDtypeStruct((K, DIM), f32),
            jax.ShapeDtypeStruct((K, K), f32),
            jax.ShapeDtypeStruct((K, DIM), f32),
        ),
        scratch_shapes=[
            pltpu.VMEM((K, DIM), jnp.bfloat16),
            pltpu.VMEM((K, HID), jnp.bfloat16),
            pltpu.VMEM((K, HID), jnp.bfloat16),
            pltpu.VMEM((K, K), f32),
            pltpu.VMEM((K, K), jnp.int32),
            pltpu.VMEM((K * K, HID), jnp.bfloat16),
            pltpu.VMEM((DIM, HID), jnp.bfloat16),
        ],
    )(u0, sb, se, tril, ssc, ssr, idxc, idxr, idxn,
      W_left, W_right, W_prod, dtp, W_dist, wo, ball, bout, wg1, wg2, bg)

    lens = jnp.broadcast_to(span_lengths.astype(jnp.int32), (16,))
    mesh = plsc.VectorSubcoreMesh(core_axis_name="c", subcore_axis_name="s",
                                  num_cores=2, num_subcores=16)
    scatter = functools.partial(
        pl.kernel,
        out_type=jax.ShapeDtypeStruct((M, DIM), f32),
        mesh=mesh,
        scratch_types=[
            pltpu.VMEM((CH, DIM), f32),
            pltpu.VMEM((CH, DIM), f32),
            pltpu.VMEM((K + 16,), jnp.int32),
            pltpu.VMEM((K + 16,), jnp.int32),
            pltpu.VMEM((K + 16,), jnp.int32),
            pltpu.VMEM((16,), jnp.int32),
            pltpu.VMEM_SHARED((K, DIM), f32),
            pltpu.SemaphoreType.DMA,
            pltpu.SemaphoreType.DMA,
            pltpu.SemaphoreType.DMA,
            pltpu.SemaphoreType.DMA,
        ],
    )(_scatter_body)
    new_cand = scatter(cand_span_vecs[0], svals, idx, lens)

    return (new_cand[None], upd[None], scores[None])
